# baseline (speedup 1.0000x reference)
import functools

import jax
import jax.numpy as jnp
from jax import lax
from jax.experimental import pallas as pl
from jax.experimental.pallas import tpu as pltpu
from jax.experimental.pallas import tpu_sc as plsc

EMB_DIM = 64
NUM_EMB = 12
LANES = 16


@functools.cache
def _expand_call(T: int, N: int):
    info = plsc.get_sparse_core_info()
    NC, NS = info.num_cores, info.num_subcores
    NW = NC * NS
    blk = N // NW
    mesh = plsc.VectorSubcoreMesh(core_axis_name="c", subcore_axis_name="s")

    @functools.partial(
        pl.kernel,
        mesh=mesh,
        compiler_params=pltpu.CompilerParams(needs_layout_passes=False),
        out_type=jax.ShapeDtypeStruct((T, EMB_DIM, N), jnp.float32),
        scratch_types=[
            pltpu.VMEM((NUM_EMB * EMB_DIM,), jnp.float32),
            pltpu.VMEM((blk,), jnp.int32),
            pltpu.VMEM((blk,), jnp.int32),
            pltpu.VMEM((EMB_DIM, blk), jnp.float32),
            pltpu.VMEM((EMB_DIM, blk), jnp.float32),
            pltpu.SemaphoreType.DMA,
            pltpu.SemaphoreType.DMA,
            pltpu.SemaphoreType.DMA,
            pltpu.SemaphoreType.DMA,
        ],
    )
    def k(tbl_hbm, catt_hbm, p_hbm,
          tbl_v, ix0, ix1, ob0, ob1, si0, si1, so0, so1):
        wid = lax.axis_index("s") * NC + lax.axis_index("c")
        b0 = wid * blk
        ixb = (ix0, ix1)
        ob = (ob0, ob1)
        si = (si0, si1)
        so = (so0, so1)

        pltpu.sync_copy(tbl_hbm, tbl_v)

        def idx_start(t, u):
            pltpu.async_copy(catt_hbm.at[t, pl.ds(b0, blk)], ixb[u], si[u])

        def idx_wait(u):
            pltpu.make_async_copy(
                catt_hbm.at[0, pl.ds(b0, blk)], ixb[u], si[u]).wait()

        def out_wait(u):
            pltpu.make_async_copy(
                ob[u], p_hbm.at[0, :, pl.ds(b0, blk)], so[u]).wait()

        idx_start(0, 0)

        def half(to, u, t):
            idx_wait(u)
            if u == 0:
                idx_start(t + 1, 1)
            else:
                @pl.when(to < T // 2 - 1)
                def _():
                    idx_start(t + 1, 0)

            @pl.when(to > 0)
            def _():
                out_wait(u)

            @plsc.parallel_loop(0, blk // LANES, unroll=4)
            def jbody(j):
                v = ixb[u][pl.ds(j * LANES, LANES)]
                f0 = v * EMB_DIM
                for d in range(EMB_DIM):
                    ob[u][d, pl.ds(j * LANES, LANES)] = plsc.load_gather(
                        tbl_v, [f0 + d])
            pltpu.async_copy(ob[u], p_hbm.at[t, :, pl.ds(b0, blk)], so[u])

        def body(to, carry):
            half(to, 0, to * 2)
            half(to, 1, to * 2 + 1)
            return carry

        lax.fori_loop(0, T // 2, body, 0)
        out_wait(0)
        out_wait(1)

    return k


def kernel(cat_ids, table):
    T = cat_ids.shape[1]
    N = cat_ids.shape[0]
    catt = cat_ids.T.astype(jnp.int32)
    p = _expand_call(T, N)(table.reshape(-1), catt)
    return jnp.transpose(p, (2, 0, 1))


# speedup vs baseline: 6.8289x; 6.8289x over previous
import functools

import jax
import jax.numpy as jnp
from jax import lax
from jax.experimental import pallas as pl
from jax.experimental.pallas import tpu as pltpu
from jax.experimental.pallas import tpu_sc as plsc

EMB_DIM = 64
NUM_EMB = 12
LANES = 16


@functools.cache
def _expand_call(T: int, N: int):
    info = plsc.get_sparse_core_info()
    NC, NS = info.num_cores, info.num_subcores
    NW = NC * NS
    blk = N // NW
    mesh = plsc.VectorSubcoreMesh(core_axis_name="c", subcore_axis_name="s")

    @functools.partial(
        pl.kernel,
        mesh=mesh,
        compiler_params=pltpu.CompilerParams(needs_layout_passes=False),
        out_type=jax.ShapeDtypeStruct((T, EMB_DIM, N), jnp.float32),
        scratch_types=[
            pltpu.VMEM((NUM_EMB * EMB_DIM * LANES,), jnp.float32),
            pltpu.VMEM((blk,), jnp.int32),
            pltpu.VMEM((blk,), jnp.int32),
            pltpu.VMEM((EMB_DIM, blk), jnp.float32),
            pltpu.VMEM((EMB_DIM, blk), jnp.float32),
            pltpu.SemaphoreType.DMA,
            pltpu.SemaphoreType.DMA,
            pltpu.SemaphoreType.DMA,
            pltpu.SemaphoreType.DMA,
        ],
    )
    def k(tbl_hbm, catt_hbm, p_hbm,
          tbl_v, ix0, ix1, ob0, ob1, si0, si1, so0, so1):
        wid = lax.axis_index("s") * NC + lax.axis_index("c")
        b0 = wid * blk
        ixb = (ix0, ix1)
        ob = (ob0, ob1)
        si = (si0, si1)
        so = (so0, so1)

        pltpu.sync_copy(tbl_hbm, tbl_v)

        def idx_start(t, u):
            pltpu.async_copy(catt_hbm.at[t, pl.ds(b0, blk)], ixb[u], si[u])

        def idx_wait(u):
            pltpu.make_async_copy(
                catt_hbm.at[0, pl.ds(b0, blk)], ixb[u], si[u]).wait()

        def out_wait(u):
            pltpu.make_async_copy(
                ob[u], p_hbm.at[0, :, pl.ds(b0, blk)], so[u]).wait()

        idx_start(0, 0)

        def half(to, u, t):
            idx_wait(u)
            if u == 0:
                idx_start(t + 1, 1)
            else:
                @pl.when(to < T // 2 - 1)
                def _():
                    idx_start(t + 1, 0)

            @pl.when(to > 0)
            def _():
                out_wait(u)

            @plsc.parallel_loop(0, blk // LANES, unroll=2)
            def jbody(j):
                v = ixb[u][pl.ds(j * LANES, LANES)]
                base = v * (EMB_DIM * LANES) + lax.iota(jnp.int32, LANES)
                for d in range(EMB_DIM):
                    ob[u][d, pl.ds(j * LANES, LANES)] = plsc.load_gather(
                        tbl_v, [base | (d * LANES)])
            pltpu.async_copy(ob[u], p_hbm.at[t, :, pl.ds(b0, blk)], so[u])

        def body(to, carry):
            half(to, 0, to * 2)
            half(to, 1, to * 2 + 1)
            return carry

        lax.fori_loop(0, T // 2, body, 0)
        out_wait(0)
        out_wait(1)

    return k


def kernel(cat_ids, table):
    T = cat_ids.shape[1]
    N = cat_ids.shape[0]
    catt = cat_ids.T.astype(jnp.int32)
    tbl_rep = jnp.tile(table.reshape(-1)[:, None], (1, LANES)).reshape(-1)
    p = _expand_call(T, N)(tbl_rep, catt)
    return jnp.transpose(p, (2, 0, 1))
